# baseline (speedup 1.0000x reference)
import functools
import jax
import jax.numpy as jnp
from jax import lax
from jax.experimental import pallas as pl
from jax.experimental.pallas import tpu as pltpu
from jax.experimental.pallas import tpu_sc as plsc

_NC, _NS, _L = 2, 16, 16
_NW = _NC * _NS
_C = 32
_SC_FRAC = 2
_BS = 512


def _sc_partial(x2d, pos, B, S, s_base, s_rows):
    D = x2d.shape[1]
    rows_per_w = s_rows // _NW
    n_chunks = rows_per_w // _C
    n_pairs = n_chunks // 2
    n_steps = n_chunks * B
    mesh = plsc.VectorSubcoreMesh(core_axis_name="c", subcore_axis_name="s")

    @functools.partial(
        pl.kernel,
        mesh=mesh,
        out_type=jax.ShapeDtypeStruct((B * S, D), jnp.float32),
        scratch_types=[
            pltpu.VMEM((_C, D), jnp.float32),
            pltpu.VMEM((_C, D), jnp.float32),
            pltpu.VMEM((_C, D), jnp.float32),
            pltpu.VMEM((_C, D), jnp.float32),
            pltpu.SemaphoreType.DMA,
            pltpu.SemaphoreType.DMA,
            pltpu.SemaphoreType.DMA,
            pltpu.SemaphoreType.DMA,
            pltpu.SemaphoreType.DMA,
            pltpu.SemaphoreType.DMA,
        ],
    )
    def k(x_hbm, pos_hbm, out_hbm, xb0, xb1, pb0, pb1,
          ls0, ls1, ss0, ss1, ps0, ps1):
        wid = lax.axis_index("s") * _NC + lax.axis_index("c")
        s0 = s_base + wid * rows_per_w
        xbufs = (xb0, xb1)
        lsems = (ls0, ls1)
        ssems = (ss0, ss1)
        pbufs = (pb0, pb1)
        psems = (ps0, ps1)

        def posrows(ci):
            return pos_hbm.at[pl.ds(s0 + ci * _C, _C)]

        def add_rows(xb, pb):
            def row_body(i, _):
                for j in range(D // _L):
                    sl = pl.ds(j * _L, _L)
                    xb[i, sl] = xb[i, sl] + pb[i, sl]
                return 0
            lax.fori_loop(0, _C, row_body, 0)

        def loop_body(cp, _):
            t0 = cp * (2 * B)
            for k_ in range(2 * B):
                t = t0 + k_
                ci_stat = k_ // B
                b = k_ % B
                ci = 2 * cp + ci_stat
                row0 = b * S + s0 + ci * _C
                xrows = x_hbm.at[pl.ds(row0, _C)]
                xb, ls, ss = xbufs[k_ % 2], lsems[k_ % 2], ssems[k_ % 2]
                xb_n, ls_n, ss_n = (xbufs[1 - k_ % 2], lsems[1 - k_ % 2],
                                    ssems[1 - k_ % 2])
                pb, ps = pbufs[ci_stat], psems[ci_stat]

                if b == 0:
                    pltpu.make_async_copy(posrows(ci), pb, ps).wait()
                    nci = ci + 1
                    npb, nps = pbufs[1 - ci_stat], psems[1 - ci_stat]
                    if ci_stat == 0:
                        pltpu.async_copy(posrows(nci), npb, nps)
                    else:
                        @pl.when(cp < n_pairs - 1)
                        def _():
                            pltpu.async_copy(posrows(nci), npb, nps)

                ci_next = 2 * cp + (k_ + 1) // B
                b_next = (k_ + 1) % B
                if k_ + 1 < 2 * B:
                    nrow0 = b_next * S + s0 + ci_next * _C
                    nxt_rows = x_hbm.at[pl.ds(nrow0, _C)]
                    @pl.when(t >= 1)
                    def _():
                        pltpu.make_async_copy(
                            xb_n, out_hbm.at[pl.ds(row0, _C)], ss_n).wait()
                    pltpu.async_copy(nxt_rows, xb_n, ls_n)
                else:
                    @pl.when(t + 1 < n_steps)
                    def _():
                        nrow0d = s0 + (2 * cp + 2) * _C
                        pltpu.make_async_copy(
                            xb_n, out_hbm.at[pl.ds(row0, _C)], ss_n).wait()
                        pltpu.async_copy(
                            x_hbm.at[pl.ds(nrow0d, _C)], xb_n, ls_n)

                pltpu.make_async_copy(xrows, xb, ls).wait()
                add_rows(xb, pb)
                pltpu.async_copy(xb, out_hbm.at[pl.ds(row0, _C)], ss)
            return 0

        pltpu.async_copy(posrows(0), pb0, ps0)
        pltpu.async_copy(x_hbm.at[pl.ds(s0, _C)], xb0, ls0)
        lax.fori_loop(0, n_pairs, loop_body, 0)
        pltpu.make_async_copy(xb0, out_hbm.at[pl.ds(s0, _C)], ss0).wait()
        pltpu.make_async_copy(xb1, out_hbm.at[pl.ds(s0, _C)], ss1).wait()

    return k(x2d, pos)


def _tc_body(x_ref, p_ref, a_ref, o_ref):
    o_ref[...] = x_ref[...] + p_ref[...]


def kernel(x, pos_table):
    B, S, D = x.shape
    s_rows = S // _SC_FRAC
    s_base = S - s_rows
    pos = pos_table[:S]

    sc_out = _sc_partial(x.reshape(B * S, D), pos, B, S, s_base, s_rows)
    sc_out = sc_out.reshape(B, S, D)

    out = pl.pallas_call(
        _tc_body,
        grid=(s_base // _BS, B),
        in_specs=[
            pl.BlockSpec((1, _BS, D), lambda s, b: (b, s, 0)),
            pl.BlockSpec((_BS, D), lambda s, b: (s, 0)),
            pl.BlockSpec(memory_space=pltpu.MemorySpace.HBM),
        ],
        out_specs=pl.BlockSpec((1, _BS, D), lambda s, b: (b, s, 0)),
        out_shape=jax.ShapeDtypeStruct((B, S, D), x.dtype),
        input_output_aliases={2: 0},
    )(x, pos, sc_out)
    return out


# speedup vs baseline: 1.0060x; 1.0060x over previous
import functools
import jax
import jax.numpy as jnp
from jax import lax
from jax.experimental import pallas as pl
from jax.experimental.pallas import tpu as pltpu
from jax.experimental.pallas import tpu_sc as plsc

_NC, _NS, _L = 2, 16, 16
_NW = _NC * _NS
_C = 32


def kernel(x, pos_table):
    B, S, D = x.shape
    rows_per_w = S // _NW
    n_chunks = rows_per_w // _C
    n_steps = n_chunks * B
    mesh = plsc.VectorSubcoreMesh(core_axis_name="c", subcore_axis_name="s")

    @functools.partial(
        pl.kernel,
        mesh=mesh,
        out_type=jax.ShapeDtypeStruct((B * S, D), jnp.float32),
        scratch_types=[
            pltpu.VMEM((_C, D), jnp.float32),
            pltpu.VMEM((_C, D), jnp.float32),
            pltpu.VMEM((_C, D), jnp.float32),
            pltpu.VMEM((_C, D), jnp.float32),
            pltpu.VMEM((_C, D), jnp.float32),
            pltpu.SemaphoreType.DMA,
            pltpu.SemaphoreType.DMA,
            pltpu.SemaphoreType.DMA,
            pltpu.SemaphoreType.DMA,
            pltpu.SemaphoreType.DMA,
            pltpu.SemaphoreType.DMA,
            pltpu.SemaphoreType.DMA,
            pltpu.SemaphoreType.DMA,
        ],
    )
    def k(x_hbm, pos_hbm, out_hbm, xb0, xb1, xb2, pb0, pb1,
          ls0, ls1, ls2, ss0, ss1, ss2, ps0, ps1):
        wid = lax.axis_index("s") * _NC + lax.axis_index("c")
        s0 = wid * rows_per_w
        xbufs = (xb0, xb1, xb2)
        lsems = (ls0, ls1, ls2)
        ssems = (ss0, ss1, ss2)
        pbufs = (pb0, pb1)
        psems = (ps0, ps1)

        def rows_of(t):
            ci, b = t // B, t % B
            return b * S + s0 + ci * _C

        def posrows(ci):
            return pos_hbm.at[pl.ds(s0 + ci * _C, _C)]

        def add_rows(xb, pb):
            def row_body(i, _):
                for j in range(D // _L):
                    sl = pl.ds(j * _L, _L)
                    xb[i, sl] = xb[i, sl] + pb[i, sl]
                return 0
            lax.fori_loop(0, _C, row_body, 0)

        pltpu.async_copy(posrows(0), pb0, ps0)
        pltpu.async_copy(x_hbm.at[pl.ds(rows_of(0), _C)], xb0, ls0)

        for t in range(n_steps):
            ci, b = t // B, t % B
            xi, pi = t % 3, ci % 2
            xb, ls, ss = xbufs[xi], lsems[xi], ssems[xi]
            pb = pbufs[pi]

            if b == 0:
                pltpu.make_async_copy(posrows(ci), pb, psems[pi]).wait()
                if ci + 1 < n_chunks:
                    pltpu.async_copy(posrows(ci + 1), pbufs[1 - pi],
                                     psems[1 - pi])

            if t + 1 < n_steps:
                ni = (t + 1) % 3
                if t >= 2:
                    pltpu.make_async_copy(
                        xbufs[ni], out_hbm.at[pl.ds(rows_of(t - 2), _C)],
                        ssems[ni]).wait()
                pltpu.async_copy(x_hbm.at[pl.ds(rows_of(t + 1), _C)],
                                 xbufs[ni], lsems[ni])

            pltpu.make_async_copy(
                x_hbm.at[pl.ds(rows_of(t), _C)], xb, ls).wait()
            add_rows(xb, pb)
            pltpu.async_copy(xb, out_hbm.at[pl.ds(rows_of(t), _C)], ss)

        for t in (n_steps - 3, n_steps - 2, n_steps - 1):
            xi = t % 3
            pltpu.make_async_copy(
                xbufs[xi], out_hbm.at[pl.ds(rows_of(t), _C)],
                ssems[xi]).wait()

    out = k(x.reshape(B * S, D), pos_table[:S])
    return out.reshape(B, S, D)


# speedup vs baseline: 1.0077x; 1.0017x over previous
import functools
import jax
import jax.numpy as jnp
from jax import lax
from jax.experimental import pallas as pl
from jax.experimental.pallas import tpu as pltpu
from jax.experimental.pallas import tpu_sc as plsc

_NC, _NS, _L = 2, 16, 16
_NW = _NC * _NS
_C = 32


def kernel(x, pos_table):
    B, S, D = x.shape
    rows_per_w = S // _NW
    n_chunks = rows_per_w // _C
    n_pairs = n_chunks // 2
    n_steps = n_chunks * B
    mesh = plsc.VectorSubcoreMesh(core_axis_name="c", subcore_axis_name="s")

    @functools.partial(
        pl.kernel,
        mesh=mesh,
        out_type=jax.ShapeDtypeStruct((B * S, D), jnp.float32),
        scratch_types=[
            pltpu.VMEM((_C, D), jnp.float32),
            pltpu.VMEM((_C, D), jnp.float32),
            pltpu.VMEM((_C, D), jnp.float32),
            pltpu.VMEM((_C, D), jnp.float32),
            pltpu.SemaphoreType.DMA,
            pltpu.SemaphoreType.DMA,
            pltpu.SemaphoreType.DMA,
            pltpu.SemaphoreType.DMA,
            pltpu.SemaphoreType.DMA,
            pltpu.SemaphoreType.DMA,
        ],
    )
    def k(x_hbm, pos_hbm, out_hbm, xb0, xb1, pb0, pb1,
          ls0, ls1, ss0, ss1, ps0, ps1):
        wid = lax.axis_index("s") * _NC + lax.axis_index("c")
        s0 = wid * rows_per_w
        xbufs = (xb0, xb1)
        lsems = (ls0, ls1)
        ssems = (ss0, ss1)
        pbufs = (pb0, pb1)
        psems = (ps0, ps1)

        def posrows(ci):
            return pos_hbm.at[pl.ds(s0 + ci * _C, _C)]

        def add_rows(xb, pb):
            def row_body(i, _):
                for j in range(D // _L):
                    sl = pl.ds(j * _L, _L)
                    xb[i, sl] = xb[i, sl] + pb[i, sl]
                return 0
            lax.fori_loop(0, _C, row_body, 0)

        def loop_body(cp, _):
            t0 = cp * (2 * B)
            for k_ in range(2 * B):
                t = t0 + k_
                ci_stat = k_ // B
                b = k_ % B
                ci = 2 * cp + ci_stat
                row0 = b * S + s0 + ci * _C
                xrows = x_hbm.at[pl.ds(row0, _C)]
                xb, ls, ss = xbufs[k_ % 2], lsems[k_ % 2], ssems[k_ % 2]
                xb_n, ls_n, ss_n = (xbufs[1 - k_ % 2], lsems[1 - k_ % 2],
                                    ssems[1 - k_ % 2])
                pb, ps = pbufs[ci_stat], psems[ci_stat]

                if b == 0:
                    pltpu.make_async_copy(posrows(ci), pb, ps).wait()
                    nci = ci + 1
                    npb, nps = pbufs[1 - ci_stat], psems[1 - ci_stat]
                    if ci_stat == 0:
                        pltpu.async_copy(posrows(nci), npb, nps)
                    else:
                        @pl.when(cp < n_pairs - 1)
                        def _():
                            pltpu.async_copy(posrows(nci), npb, nps)

                ci_next = 2 * cp + (k_ + 1) // B
                b_next = (k_ + 1) % B
                if k_ + 1 < 2 * B:
                    nrow0 = b_next * S + s0 + ci_next * _C
                    nxt_rows = x_hbm.at[pl.ds(nrow0, _C)]
                    @pl.when(t >= 1)
                    def _():
                        pltpu.make_async_copy(
                            xb_n, out_hbm.at[pl.ds(row0, _C)], ss_n).wait()
                    pltpu.async_copy(nxt_rows, xb_n, ls_n)
                else:
                    @pl.when(t + 1 < n_steps)
                    def _():
                        nrow0d = s0 + (2 * cp + 2) * _C
                        pltpu.make_async_copy(
                            xb_n, out_hbm.at[pl.ds(row0, _C)], ss_n).wait()
                        pltpu.async_copy(
                            x_hbm.at[pl.ds(nrow0d, _C)], xb_n, ls_n)

                pltpu.make_async_copy(xrows, xb, ls).wait()
                add_rows(xb, pb)
                pltpu.async_copy(xb, out_hbm.at[pl.ds(row0, _C)], ss)
            return 0

        pltpu.async_copy(posrows(0), pb0, ps0)
        pltpu.async_copy(x_hbm.at[pl.ds(s0, _C)], xb0, ls0)
        lax.fori_loop(0, n_pairs, loop_body, 0)
        pltpu.make_async_copy(xb0, out_hbm.at[pl.ds(s0, _C)], ss0).wait()
        pltpu.make_async_copy(xb1, out_hbm.at[pl.ds(s0, _C)], ss1).wait()

    out = k(x.reshape(B * S, D), pos_table[:S])
    return out.reshape(B, S, D)
